# baseline (speedup 1.0000x reference)
import functools

import jax
import jax.numpy as jnp
from jax.experimental import pallas as pl
from jax.experimental.pallas import tpu as pltpu

_SMOOTH = 1e-06


def _dice_body(B, C, pred_ref, tgt_ref, out_ref, acc_i, acc_s, acc_c,
               dsum_ref):
    b = pl.program_id(0)

    @pl.when(b == 0)
    def _init_scalar():
        dsum_ref[0] = 0.0

    H = tgt_ref.shape[1]

    def _tree128(x):
        return (x[:, 0:128] + x[:, 128:256]) + (x[:, 256:384] + x[:, 384:512])

    zi = jnp.zeros((8, 128), jnp.float32)
    for c in range(C):
        ai = zi
        ac = zi
        asum = zi
        for k in range(H // 8):
            tk = tgt_ref[0, k * 8:(k + 1) * 8, :].astype(jnp.int32)
            pk = pred_ref[0, c, k * 8:(k + 1) * 8, :]
            m = tk == c
            ai = ai + _tree128(jnp.where(m, pk, 0.0))
            ac = ac + _tree128(jnp.where(m, 1.0, 0.0))
            asum = asum + _tree128(pk)
        acc_i[c, :, :] = ai
        acc_c[c, :, :] = ac
        acc_s[c, :, :] = asum

    total = dsum_ref[0]
    for c in range(C):
        inter = jnp.sum(acc_i[c, :, :])
        cnt = jnp.sum(acc_c[c, :, :])
        psum = jnp.sum(acc_s[c, :, :])
        total += (2.0 * inter + _SMOOTH) / (psum + cnt + _SMOOTH)
    dsum_ref[0] = total

    @pl.when(b == B - 1)
    def _emit():
        out_ref[0] = 1.0 - dsum_ref[0] / (B * C)


def kernel(pred, target):
    B, C, H, W = pred.shape
    target = target.astype(jnp.int8)

    body = functools.partial(_dice_body, B, C)

    out = pl.pallas_call(
        body,
        grid=(B,),
        in_specs=[
            pl.BlockSpec((1, C, H, W), lambda b: (b, 0, 0, 0)),
            pl.BlockSpec((1, H, W), lambda b: (b, 0, 0)),
        ],
        out_specs=pl.BlockSpec(memory_space=pltpu.SMEM),
        out_shape=jax.ShapeDtypeStruct((1,), jnp.float32),
        scratch_shapes=[
            pltpu.VMEM((C, 8, 128), jnp.float32),
            pltpu.VMEM((C, 8, 128), jnp.float32),
            pltpu.VMEM((C, 8, 128), jnp.float32),
            pltpu.SMEM((1,), jnp.float32),
        ],
        compiler_params=pltpu.CompilerParams(
            dimension_semantics=("arbitrary",)),
    )(pred, target)
    return out[0]


# speedup vs baseline: 1.1352x; 1.1352x over previous
import functools

import jax
import jax.numpy as jnp
from jax.experimental import pallas as pl
from jax.experimental.pallas import tpu as pltpu

_SMOOTH = 1e-06


def _dice_body(B, C, pred_ref, tgt_ref, out_ref, acc_i, acc_s, acc_c,
               dsum_ref):
    b = pl.program_id(0)

    @pl.when(b == 0)
    def _init_scalar():
        dsum_ref[0] = 0.0

    H = tgt_ref.shape[1]

    def _tree128(x):
        return (x[:, 0:128] + x[:, 128:256]) + (x[:, 256:384] + x[:, 384:512])

    zi = jnp.zeros((8, 128), jnp.float32)
    for c in range(C):
        ai = zi
        ac = zi
        asum = zi
        for k in range(H // 8):
            tk = tgt_ref[0, k * 8:(k + 1) * 8, :]
            pk = pred_ref[0, c, k * 8:(k + 1) * 8, :]
            m = tk == c
            ai = ai + _tree128(jnp.where(m, pk, 0.0))
            ac = ac + _tree128(jnp.where(m, 1.0, 0.0))
            asum = asum + _tree128(pk)
        acc_i[c, :, :] = ai
        acc_c[c, :, :] = ac
        acc_s[c, :, :] = asum

    total = dsum_ref[0]
    for c in range(C):
        inter = jnp.sum(acc_i[c, :, :])
        cnt = jnp.sum(acc_c[c, :, :])
        psum = jnp.sum(acc_s[c, :, :])
        total += (2.0 * inter + _SMOOTH) / (psum + cnt + _SMOOTH)
    dsum_ref[0] = total

    @pl.when(b == B - 1)
    def _emit():
        out_ref[0] = 1.0 - dsum_ref[0] / (B * C)


def kernel(pred, target):
    B, C, H, W = pred.shape

    body = functools.partial(_dice_body, B, C)

    out = pl.pallas_call(
        body,
        grid=(B,),
        in_specs=[
            pl.BlockSpec((1, C, H, W), lambda b: (b, 0, 0, 0)),
            pl.BlockSpec((1, H, W), lambda b: (b, 0, 0)),
        ],
        out_specs=pl.BlockSpec(memory_space=pltpu.SMEM),
        out_shape=jax.ShapeDtypeStruct((1,), jnp.float32),
        scratch_shapes=[
            pltpu.VMEM((C, 8, 128), jnp.float32),
            pltpu.VMEM((C, 8, 128), jnp.float32),
            pltpu.VMEM((C, 8, 128), jnp.float32),
            pltpu.SMEM((1,), jnp.float32),
        ],
        compiler_params=pltpu.CompilerParams(
            dimension_semantics=("arbitrary",)),
    )(pred, target)
    return out[0]
